# baseline (device time: 200351 ns/iter reference)
import jax
import jax.numpy as jnp
from jax import lax
from jax.experimental import pallas as pl
from jax.experimental.pallas import tpu as pltpu

N_DEV = 32
M = 4096
M_PER = M // N_DEV
N_COLS = 2048
N_HALF = N_COLS // 2
N_SUB = N_HALF // 2
N_STEPS = N_DEV - 1
LOG2_DEV = 5


def _ring_tables():
    logical = []
    for z in range(4):
        for y in range(4):
            for x in ((0, 1) if y % 2 == 0 else (1, 0)):
                logical.append((x, y, z))
    bous = [(0, 0), (1, 0), (2, 0), (3, 0), (3, 1), (2, 1), (1, 1), (0, 1),
            (0, 2), (1, 2), (2, 2), (3, 2), (3, 3), (2, 3), (1, 3), (0, 3)]
    cycle = [(0, y, z) for (y, z) in bous] + \
            [(1, y, z) for (y, z) in reversed(bous)]
    sigma = [logical.index(c) for c in cycle]
    inv = [0] * N_DEV
    for r, l in enumerate(sigma):
        inv[l] = r
    flips = [(1, 0, 0), (0, 1, 0), (0, 2, 0), (0, 0, 1), (0, 0, 2)]
    pbit = []
    for fx, fy, fz in flips:
        row = []
        for (cx, cy, cz) in logical:
            row.append(logical.index((cx ^ fx, cy ^ fy, cz ^ fz)))
        pbit.append(row)
    return sigma, inv, pbit


_SIGMA, _INV, _PBIT = _ring_tables()


def kernel(x, w_mat):
    def body(sigma_ref, inv_ref, pbit_ref, x_ref, w_ref, out_ref,
             send_r, recv_r, send_l, recv_l,
             send_sems_r, recv_sems_r, send_sems_l, recv_sems_l,
             creditA_r, creditB_r, creditA_l, creditB_l,
             amax_send, amax_recv, amax_send_sems, amax_recv_sems):
        d = lax.axis_index("i")
        rho = inv_ref[d]
        right = sigma_ref[jnp.mod(rho + 1, N_DEV)]
        left = sigma_ref[jnp.mod(rho - 1, N_DEV)]

        barrier_sem = pltpu.get_barrier_semaphore()
        for nbr in (left, right):
            pl.semaphore_signal(barrier_sem, inc=1, device_id=(nbr,),
                                device_id_type=pl.DeviceIdType.MESH)
        pl.semaphore_wait(barrier_sem, 2)

        def partial(c, lo):
            return jnp.dot(
                x_ref[pl.ds(c * M_PER, M_PER), :],
                w_ref[:, lo:lo + N_SUB],
                preferred_element_type=jnp.float32,
            )

        dirs = {
            "r": (send_r, recv_r, send_sems_r, recv_sems_r, right, left, 0),
            "l": (send_l, recv_l, send_sems_l, recv_sems_l, left, right, N_HALF),
        }
        credits = {("r", 0): creditA_r, ("r", 1): creditB_r,
                   ("l", 0): creditA_l, ("l", 1): creditB_l}

        def chunk_id(dirname, s):
            if dirname == "r":
                return sigma_ref[jnp.mod(rho - 2 - s, N_DEV)]
            return sigma_ref[jnp.mod(rho + 2 + s, N_DEV)]

        def make_rdma(dirname, sub, slot):
            sb, rb, ss, rs, peer_out, _, _ = dirs[dirname]
            return pltpu.make_async_remote_copy(
                src_ref=sb.at[sub, slot], dst_ref=rb.at[sub, slot],
                send_sem=ss.at[sub, slot], recv_sem=rs.at[sub, slot],
                device_id=(peer_out,), device_id_type=pl.DeviceIdType.MESH)

        last_send = {}

        for dirname in ("r", "l"):
            sb = dirs[dirname][0]
            base = dirs[dirname][6]
            c0 = (sigma_ref[jnp.mod(rho - 1, N_DEV)] if dirname == "r"
                  else sigma_ref[jnp.mod(rho + 1, N_DEV)])
            for sub in (0, 1):
                sb[sub, 0, :, :] = partial(c0, base + sub * N_SUB)
        for dirname in ("r", "l"):
            for sub in (0, 1):
                rd = make_rdma(dirname, sub, 0)
                rd.start()
                last_send[(dirname, sub, 0)] = rd

        y = {}
        for s in range(N_STEPS):
            sp = s % 2
            nsp = (s + 1) % 2
            last = s == N_STEPS - 1
            p = {}
            for dirname in ("r", "l"):
                base = dirs[dirname][6]
                c = chunk_id(dirname, s)
                for sub in (0, 1):
                    p[(dirname, sub)] = partial(c, base + sub * N_SUB)

            for sub in (0, 1):
                for dirname in ("r", "l"):
                    sb, rb, ss, rs, peer_out, peer_in, base = dirs[dirname]
                    make_rdma(dirname, sub, sp).wait_recv()
                    acc = rb[sub, sp, :, :] + p[(dirname, sub)]
                    if not last:
                        prev = last_send.get((dirname, sub, nsp))
                        if prev is not None:
                            prev.wait_send()
                        sb[sub, nsp, :, :] = acc
                        if s >= 1:
                            pl.semaphore_wait(credits[(dirname, sub)], 1)
                        rd = make_rdma(dirname, sub, nsp)
                        rd.start()
                        last_send[(dirname, sub, nsp)] = rd
                        if s <= N_STEPS - 3:
                            pl.semaphore_signal(
                                credits[(dirname, sub)], inc=1,
                                device_id=(peer_in,),
                                device_id_type=pl.DeviceIdType.MESH)
                    else:
                        y[(dirname, sub)] = jnp.maximum(acc, 0.0)

        for key, rd in last_send.items():
            rd.wait_send()

        amax = jnp.maximum(
            jnp.maximum(jnp.max(y[("r", 0)]), jnp.max(y[("r", 1)])),
            jnp.maximum(jnp.max(y[("l", 0)]), jnp.max(y[("l", 1)])))
        for r in range(LOG2_DEV):
            partner = pbit_ref[r, d]
            amax_send[r, :, :] = jnp.full((8, 128), amax, jnp.float32)
            ex = pltpu.make_async_remote_copy(
                src_ref=amax_send.at[r], dst_ref=amax_recv.at[r],
                send_sem=amax_send_sems.at[r], recv_sem=amax_recv_sems.at[r],
                device_id=(partner,), device_id_type=pl.DeviceIdType.MESH)
            ex.start()
            ex.wait()
            amax = jnp.maximum(amax, jnp.max(amax_recv[r, :, :]))

        scale = amax / 127.0
        inv_scale = 127.0 / amax
        for (dirname, sub), yv in y.items():
            base = dirs[dirname][6] + sub * N_SUB
            q = jnp.clip(jnp.round(yv * inv_scale), 0.0, 127.0)
            out_ref[:, base:base + N_SUB] = q * scale

    sigma_arr = jnp.array(_SIGMA, dtype=jnp.int32)
    inv_arr = jnp.array(_INV, dtype=jnp.int32)
    pbit_arr = jnp.array(_PBIT, dtype=jnp.int32)

    return pl.pallas_call(
        body,
        out_shape=jax.ShapeDtypeStruct((M_PER, N_COLS), jnp.float32),
        in_specs=[
            pl.BlockSpec(memory_space=pltpu.SMEM),
            pl.BlockSpec(memory_space=pltpu.SMEM),
            pl.BlockSpec(memory_space=pltpu.SMEM),
            pl.BlockSpec(memory_space=pltpu.VMEM),
            pl.BlockSpec(memory_space=pltpu.VMEM),
        ],
        out_specs=pl.BlockSpec(memory_space=pltpu.VMEM),
        scratch_shapes=[
            pltpu.VMEM((2, 2, M_PER, N_SUB), jnp.float32),
            pltpu.VMEM((2, 2, M_PER, N_SUB), jnp.float32),
            pltpu.VMEM((2, 2, M_PER, N_SUB), jnp.float32),
            pltpu.VMEM((2, 2, M_PER, N_SUB), jnp.float32),
            pltpu.SemaphoreType.DMA((2, 2)),
            pltpu.SemaphoreType.DMA((2, 2)),
            pltpu.SemaphoreType.DMA((2, 2)),
            pltpu.SemaphoreType.DMA((2, 2)),
            pltpu.SemaphoreType.REGULAR,
            pltpu.SemaphoreType.REGULAR,
            pltpu.SemaphoreType.REGULAR,
            pltpu.SemaphoreType.REGULAR,
            pltpu.VMEM((LOG2_DEV, 8, 128), jnp.float32),
            pltpu.VMEM((LOG2_DEV, 8, 128), jnp.float32),
            pltpu.SemaphoreType.DMA((LOG2_DEV,)),
            pltpu.SemaphoreType.DMA((LOG2_DEV,)),
        ],
        compiler_params=pltpu.CompilerParams(collective_id=0),
    )(sigma_arr, inv_arr, pbit_arr, x, w_mat)
